# baseline (speedup 1.0000x reference)
import jax
import jax.numpy as jnp
from jax.experimental import pallas as pl
from jax.experimental.pallas import tpu as pltpu

N = 4096
_BM = 512
_NB = N // _BM
_RM = 1024
_F32 = jnp.float32
_BF16 = jnp.bfloat16


def _mega_kernel(x_ref, adj_ref, w1, w2, w3, w4, w5, w6,
                 zs_out, zh_out, zsb_out, mb_out, mg_out,
                 adjb, ta, tb, pe, pd, pg):
    l = pl.program_id(0)
    i = pl.program_id(1)
    rows = pl.ds(i * _BM, _BM)

    def dot(a, b):
        return jnp.dot(a, b, preferred_element_type=_F32)

    @pl.when(l == 0)
    def _():
        @pl.when(i == 0)
        def _():
            e = dot(dot(w1[...], w2[...]), w3[...])
            pe[...] = e.astype(_BF16)
            d = dot(dot(w4[...], w5[...]), w6[...])
            pd[...] = d.astype(_BF16)
            g = jax.lax.dot_general(d, d, (((1,), (1,)), ((), ())),
                                    preferred_element_type=_F32)
            pg[...] = (0.5 * g).astype(_BF16)

        ta[rows, :] = dot(x_ref[...], pe[...]).astype(_BF16)

    @pl.when(l == 1)
    def _():
        ab = adj_ref[...].astype(_BF16)
        adjb[rows, :] = ab
        tb[rows, :] = dot(ab, ta[...]).astype(_BF16)

    @pl.when(l == 2)
    def _():
        ta[rows, :] = dot(adjb[rows, :], tb[...]).astype(_BF16)

    @pl.when(l == 3)
    def _():
        acc = dot(adjb[rows, :], ta[...])
        zs_out[...] = acc
        zb = acc.astype(_BF16)
        zsb_out[...] = zb
        tb[rows, :] = zb

    @pl.when(l == 4)
    def _():
        ta[rows, :] = dot(adjb[rows, :], tb[...]).astype(_BF16)

    @pl.when(l == 5)
    def _():
        tb[rows, :] = dot(adjb[rows, :], ta[...]).astype(_BF16)

    @pl.when(l == 6)
    def _():
        m = dot(adjb[rows, :], tb[...])
        mb = m.astype(_BF16)
        zh_out[...] = dot(mb, pd[...])
        mb_out[...] = mb
        mg_out[...] = dot(mb, pg[...]).astype(_BF16)


def _recon_kernel(zs_i, zs_all, mg_i, m_all, o_ref):
    dn = (((1,), (1,)), ((), ()))
    a = jax.lax.dot_general(zs_i[...] * jnp.bfloat16(0.5), zs_all[...], dn,
                            preferred_element_type=_F32)
    b = jax.lax.dot_general(mg_i[...], m_all[...], dn,
                            preferred_element_type=_F32)
    o_ref[...] = 0.5 * (jnp.tanh(a) + jnp.tanh(b)) + 1.0


def _recon(zs, mg, m):
    return pl.pallas_call(
        _recon_kernel,
        grid=(N // _RM,),
        in_specs=[
            pl.BlockSpec((_RM, zs.shape[1]), lambda i: (i, 0)),
            pl.BlockSpec(zs.shape, lambda i: (0, 0)),
            pl.BlockSpec((_RM, mg.shape[1]), lambda i: (i, 0)),
            pl.BlockSpec(m.shape, lambda i: (0, 0)),
        ],
        out_specs=pl.BlockSpec((_RM, N), lambda i: (i, 0)),
        out_shape=jax.ShapeDtypeStruct((N, N), _F32),
    )(zs, zs, mg, m)


def kernel(x, adj, W1, W2, W3, W4, W5, W6):
    last = _NB - 1

    def _x_map(l, i):
        return (jnp.where(l == 0, i, last), 0)

    def _adj_map(l, i):
        return (jnp.where(l == 1, i, jnp.where(l < 1, 0, last)), 0)

    def _w_map(l, i):
        return (0, 0)

    def _zs_map(l, i):
        return (jnp.where(l == 3, i, jnp.where(l < 3, 0, last)), 0)

    def _zh_map(l, i):
        return (jnp.where(l == 6, i, 0), 0)

    ws = (W1, W2, W3, W4, W5, W6)
    in_specs = [
        pl.BlockSpec((_BM, x.shape[1]), _x_map),
        pl.BlockSpec((_BM, N), _adj_map),
    ]
    for w in ws:
        in_specs.append(pl.BlockSpec(w.shape, _w_map))

    z_sgae, z_hat, zsb16, mb16, mg16 = pl.pallas_call(
        _mega_kernel,
        grid=(7, _NB),
        in_specs=in_specs,
        out_specs=(pl.BlockSpec((_BM, 128), _zs_map),
                   pl.BlockSpec((_BM, 512), _zh_map),
                   pl.BlockSpec((_BM, 128), _zs_map),
                   pl.BlockSpec((_BM, 128), _zh_map),
                   pl.BlockSpec((_BM, 128), _zh_map)),
        out_shape=(jax.ShapeDtypeStruct((N, 128), _F32),
                   jax.ShapeDtypeStruct((N, 512), _F32),
                   jax.ShapeDtypeStruct((N, 128), _BF16),
                   jax.ShapeDtypeStruct((N, 128), _BF16),
                   jax.ShapeDtypeStruct((N, 128), _BF16)),
        scratch_shapes=[
            pltpu.VMEM((N, N), _BF16),
            pltpu.VMEM((N, 128), _BF16),
            pltpu.VMEM((N, 128), _BF16),
            pltpu.VMEM((512, 128), _BF16),
            pltpu.VMEM((128, 512), _BF16),
            pltpu.VMEM((128, 128), _BF16),
        ],
    )(x.astype(_BF16), adj, *ws)

    adj_hat = _recon(zsb16, mg16, mb16)
    return (z_sgae, z_hat, adj_hat)


# speedup vs baseline: 1.0517x; 1.0517x over previous
import jax
import jax.numpy as jnp
from jax.experimental import pallas as pl
from jax.experimental.pallas import tpu as pltpu

N = 4096
_BM = 512
_NB = N // _BM
_RM = 512
_F32 = jnp.float32
_BF16 = jnp.bfloat16


def _mega_kernel(x_ref, adj_ref, w1, w2, w3, w4, w5, w6,
                 zs_out, zh_out, zsb_out, mb_out, mg_out,
                 adjb, ta, tb, pe, pd, pg):
    l = pl.program_id(0)
    i = pl.program_id(1)
    rows = pl.ds(i * _BM, _BM)

    def dot(a, b):
        return jnp.dot(a, b, preferred_element_type=_F32)

    @pl.when(l == 0)
    def _():
        @pl.when(i == 0)
        def _():
            e = dot(dot(w1[...], w2[...]), w3[...])
            pe[...] = e.astype(_BF16)
            d = dot(dot(w4[...], w5[...]), w6[...])
            pd[...] = d.astype(_BF16)
            g = jax.lax.dot_general(d, d, (((1,), (1,)), ((), ())),
                                    preferred_element_type=_F32)
            pg[...] = (0.5 * g).astype(_BF16)

        ta[rows, :] = dot(x_ref[...].astype(_BF16), pe[...]).astype(_BF16)

    @pl.when(l == 1)
    def _():
        ab = adj_ref[...].astype(_BF16)
        adjb[rows, :] = ab
        tb[rows, :] = dot(ab, ta[...]).astype(_BF16)

    @pl.when(l == 2)
    def _():
        ta[rows, :] = dot(adjb[rows, :], tb[...]).astype(_BF16)

    @pl.when(l == 3)
    def _():
        acc = dot(adjb[rows, :], ta[...])
        zs_out[...] = acc
        zb = acc.astype(_BF16)
        zsb_out[...] = zb
        tb[rows, :] = zb

    @pl.when(l == 4)
    def _():
        ta[rows, :] = dot(adjb[rows, :], tb[...]).astype(_BF16)

    @pl.when(l == 5)
    def _():
        tb[rows, :] = dot(adjb[rows, :], ta[...]).astype(_BF16)

    @pl.when(l == 6)
    def _():
        m = dot(adjb[rows, :], tb[...])
        mb = m.astype(_BF16)
        zh_out[...] = dot(mb, pd[...])
        mb_out[...] = mb
        mg_out[...] = dot(mb, pg[...]).astype(_BF16)


def _recon_kernel(zs_i, zs_all, mg_i, m_all, o_ref):
    dn = (((1,), (1,)), ((), ()))
    a = jax.lax.dot_general(zs_i[...] * jnp.bfloat16(0.5), zs_all[...], dn,
                            preferred_element_type=_F32)
    b = jax.lax.dot_general(mg_i[...], m_all[...], dn,
                            preferred_element_type=_F32)
    o_ref[...] = 0.5 * (jnp.tanh(a) + jnp.tanh(b)) + 1.0


def _recon(zs, mg, m):
    return pl.pallas_call(
        _recon_kernel,
        grid=(N // _RM,),
        in_specs=[
            pl.BlockSpec((_RM, zs.shape[1]), lambda i: (i, 0)),
            pl.BlockSpec(zs.shape, lambda i: (0, 0)),
            pl.BlockSpec((_RM, mg.shape[1]), lambda i: (i, 0)),
            pl.BlockSpec(m.shape, lambda i: (0, 0)),
        ],
        out_specs=pl.BlockSpec((_RM, N), lambda i: (i, 0)),
        out_shape=jax.ShapeDtypeStruct((N, N), _F32),
    )(zs, zs, mg, m)


def kernel(x, adj, W1, W2, W3, W4, W5, W6):
    last = _NB - 1

    def _x_map(l, i):
        return (jnp.where(l == 0, i, last), 0)

    def _adj_map(l, i):
        return (jnp.where(l == 1, i, jnp.where(l < 1, 0, last)), 0)

    def _w_map(l, i):
        return (0, 0)

    def _zs_map(l, i):
        return (jnp.where(l == 3, i, jnp.where(l < 3, 0, last)), 0)

    def _zh_map(l, i):
        return (jnp.where(l == 6, i, 0), 0)

    ws = (W1, W2, W3, W4, W5, W6)
    in_specs = [
        pl.BlockSpec((_BM, x.shape[1]), _x_map),
        pl.BlockSpec((_BM, N), _adj_map),
    ]
    for w in ws:
        in_specs.append(pl.BlockSpec(w.shape, _w_map))

    z_sgae, z_hat, zsb16, mb16, mg16 = pl.pallas_call(
        _mega_kernel,
        grid=(7, _NB),
        in_specs=in_specs,
        out_specs=(pl.BlockSpec((_BM, 128), _zs_map),
                   pl.BlockSpec((_BM, 512), _zh_map),
                   pl.BlockSpec((_BM, 128), _zs_map),
                   pl.BlockSpec((_BM, 128), _zh_map),
                   pl.BlockSpec((_BM, 128), _zh_map)),
        out_shape=(jax.ShapeDtypeStruct((N, 128), _F32),
                   jax.ShapeDtypeStruct((N, 512), _F32),
                   jax.ShapeDtypeStruct((N, 128), _BF16),
                   jax.ShapeDtypeStruct((N, 128), _BF16),
                   jax.ShapeDtypeStruct((N, 128), _BF16)),
        scratch_shapes=[
            pltpu.VMEM((N, N), _BF16),
            pltpu.VMEM((N, 128), _BF16),
            pltpu.VMEM((N, 128), _BF16),
            pltpu.VMEM((512, 128), _BF16),
            pltpu.VMEM((128, 512), _BF16),
            pltpu.VMEM((128, 128), _BF16),
        ],
        compiler_params=pltpu.CompilerParams(
            vmem_limit_bytes=100 * 1024 * 1024),
    )(x, adj, *ws)

    adj_hat = _recon(zsb16, mg16, mb16)
    return (z_sgae, z_hat, adj_hat)
